# Initial kernel scaffold; baseline (speedup 1.0000x reference)
import jax
import jax.numpy as jnp
from jax.experimental import pallas as pl


def kernel(center_words, context_words, negative_words, W_center, W_context):
    raise NotImplementedError("write your pallas kernel here")


# speedup vs baseline: 4.7581x; 4.7581x over previous
import functools

import jax
import jax.numpy as jnp
from jax import lax
from jax.experimental import pallas as pl
from jax.experimental.pallas import tpu as pltpu
from jax.experimental.pallas import tpu_sc as plsc

_NC = 2
_NS = 16
_NW = _NC * _NS
_L = 16


def _sc_scores(cw, xw, nw_flat, Wc, Wx, B, N, D):
    bpw = B // _NW
    CH = _L
    n_chunks = bpw // CH
    NIDX = CH * N
    DV = D // _L

    mesh = plsc.VectorSubcoreMesh(core_axis_name="c", subcore_axis_name="s")

    @functools.partial(
        pl.kernel, mesh=mesh,
        compiler_params=pltpu.CompilerParams(
            needs_layout_passes=False, use_tc_tiling_on_sc=False),
        out_type=(jax.ShapeDtypeStruct((B,), jnp.float32),
                  jax.ShapeDtypeStruct((B * N,), jnp.float32)),
        scratch_types=[
            pltpu.VMEM((CH,), jnp.int32),
            pltpu.VMEM((CH,), jnp.int32),
            pltpu.VMEM((NIDX,), jnp.int32),
            pltpu.VMEM((CH, D), jnp.float32),
            pltpu.VMEM((CH, D), jnp.float32),
            pltpu.VMEM((NIDX, D), jnp.float32),
            pltpu.VMEM((CH,), jnp.float32),
            pltpu.VMEM((NIDX,), jnp.float32),
            pltpu.SemaphoreType.DMA,
        ],
    )
    def k(cw_hbm, xw_hbm, nw_hbm, Wc_hbm, Wx_hbm, pos_hbm, neg_hbm,
          cidx, xidx, nidx, cbuf, xbuf, nbuf, posb, negb, sem):
        wid = lax.axis_index("s") * _NC + lax.axis_index("c")
        base = wid * bpw
        lanes = lax.iota(jnp.int32, _L)

        def chunk_body(g, carry):
            goff = base + g * CH
            pltpu.sync_copy(cw_hbm.at[pl.ds(goff, CH)], cidx)
            pltpu.sync_copy(xw_hbm.at[pl.ds(goff, CH)], xidx)
            pltpu.sync_copy(nw_hbm.at[pl.ds(goff * N, NIDX)], nidx)
            cps = [pltpu.async_copy(Wc_hbm.at[cidx], cbuf, sem),
                   pltpu.async_copy(Wx_hbm.at[xidx], xbuf, sem)]
            j = 0
            while j < NIDX:
                w = min(128, NIDX - j)
                cps.append(pltpu.async_copy(
                    Wx_hbm.at[nidx.at[pl.ds(j, w)]],
                    nbuf.at[pl.ds(j, w)], sem))
                j += w
            for cp in cps:
                cp.wait()

            pv = jnp.zeros((_L,), jnp.float32)
            nvecs = [jnp.zeros((_L,), jnp.float32) for _ in range(NIDX // _L)]
            for e in range(CH):
                c = [cbuf[e, pl.ds(k2 * _L, _L)] for k2 in range(DV)]
                x = [xbuf[e, pl.ds(k2 * _L, _L)] for k2 in range(DV)]
                acc = c[0] * x[0]
                for k2 in range(1, DV):
                    acc = acc + c[k2] * x[k2]
                pv = jnp.where(lanes == e, jnp.sum(acc), pv)
                for n in range(N):
                    r = e * N + n
                    y = [nbuf[r, pl.ds(k2 * _L, _L)] for k2 in range(DV)]
                    a = c[0] * y[0]
                    for k2 in range(1, DV):
                        a = a + c[k2] * y[k2]
                    nvecs[r // _L] = jnp.where(
                        lanes == (r % _L), jnp.sum(a), nvecs[r // _L])
            posb[...] = pv
            for v in range(NIDX // _L):
                negb[pl.ds(v * _L, _L)] = nvecs[v]
            pltpu.sync_copy(posb, pos_hbm.at[pl.ds(goff, CH)])
            pltpu.sync_copy(negb, neg_hbm.at[pl.ds(goff * N, NIDX)])
            return carry

        lax.fori_loop(0, n_chunks, chunk_body, 0)

    return k(cw, xw, nw_flat, Wc, Wx)


def _tc_loss(pos2d, neg2d, B):
    def body(pos_ref, neg_ref, out_ref):
        def ls(x):
            return jnp.minimum(x, 0.0) - jnp.log1p(jnp.exp(-jnp.abs(x)))
        tot = jnp.sum(ls(pos_ref[...])) + jnp.sum(ls(-neg_ref[...]))
        out_ref[0, 0] = -tot / B

    return pl.pallas_call(
        body,
        out_shape=jax.ShapeDtypeStruct((1, 1), jnp.float32),
        out_specs=pl.BlockSpec(memory_space=pltpu.SMEM),
    )(pos2d, neg2d)


def kernel(center_words, context_words, negative_words, W_center, W_context):
    B, N = negative_words.shape
    D = W_center.shape[1]
    cw = center_words.astype(jnp.int32)
    xw = context_words.astype(jnp.int32)
    nw = negative_words.astype(jnp.int32).reshape(B * N)
    pos, neg = _sc_scores(cw, xw, nw, W_center, W_context, B, N, D)
    loss = _tc_loss(pos.reshape(B // 128, 128), neg.reshape(B * N // 128, 128), B)
    return loss[0, 0]
